# baseline (speedup 1.0000x reference)
import functools

import jax
import jax.numpy as jnp
from jax import lax
from jax.experimental import pallas as pl
from jax.experimental.pallas import tpu as pltpu
from jax.experimental.pallas import tpu_sc as plsc

D = 128
W = D // 4
L = 16

_info = plsc.get_sparse_core_info()
NC, NS = _info.num_cores, _info.num_subcores
NW = NC * NS

CHUNK = 128
PCHUNK = 160

_SC_PARAMS = pltpu.CompilerParams(use_tc_tiling_on_sc=False)


def _make_pack(V):
    n_chunks = -(-V // PCHUNK)
    assert V % PCHUNK == 0
    mesh = plsc.VectorSubcoreMesh(core_axis_name="c", subcore_axis_name="s")

    @functools.partial(
        pl.kernel,
        mesh=mesh,
        out_type=jax.ShapeDtypeStruct((V, W), jnp.int32),
        compiler_params=_SC_PARAMS,
        scratch_types=[
            pltpu.VMEM((PCHUNK, D), jnp.float32),
            pltpu.VMEM((PCHUNK, D), jnp.float32),
            pltpu.VMEM((PCHUNK, W), jnp.int32),
            pltpu.VMEM((PCHUNK, W), jnp.int32),
            pltpu.SemaphoreType.DMA,
            pltpu.SemaphoreType.DMA,
            pltpu.SemaphoreType.DMA,
            pltpu.SemaphoreType.DMA,
        ],
    )
    def pack_k(tab_hbm, out_hbm, t0, t1, p0, p1, sg0, sg1, sw0, sw1):
        tbuf, pbuf = (t0, t1), (p0, p1)
        sg, sw = (sg0, sg1), (sw0, sw1)
        wid = lax.axis_index("s") * NC + lax.axis_index("c")
        per_tile = -(-n_chunks // NW)

        def chunk_rows(i):
            c = jnp.minimum(wid + i * NW, n_chunks - 1)
            return c * PCHUNK

        pltpu.async_copy(tab_hbm.at[pl.ds(chunk_rows(0), PCHUNK)], tbuf[0],
                         sg[0])
        pltpu.async_copy(tab_hbm.at[pl.ds(chunk_rows(1), PCHUNK)], tbuf[1],
                         sg[1])

        def encode(src, dst):
            def row(r, carry):
                for v in range(2):
                    word = None
                    for b in range(4):
                        x = src[r, pl.ds(64 * v + 16 * b, L)]
                        s = jnp.where(x >= 0, jnp.int32(1 << (8 * b)),
                                      jnp.int32(0))
                        word = s if word is None else word | s
                    dst[r, pl.ds(16 * v, L)] = word
                return carry
            lax.fori_loop(0, PCHUNK, row, 0)

        def step(i, carry):
            for b in range(2):
                g = 2 * i + b
                rows = chunk_rows(g)
                pltpu.make_async_copy(
                    tab_hbm.at[pl.ds(rows, PCHUNK)], tbuf[b], sg[b]).wait()
                @pl.when(g >= 2)
                def _():
                    pltpu.make_async_copy(
                        pbuf[b], out_hbm.at[pl.ds(rows, PCHUNK)], sw[b]).wait()
                encode(tbuf[b], pbuf[b])
                @pl.when(g + 2 < per_tile)
                def _():
                    pltpu.async_copy(
                        tab_hbm.at[pl.ds(chunk_rows(g + 2), PCHUNK)],
                        tbuf[b], sg[b])
                pltpu.async_copy(
                    pbuf[b], out_hbm.at[pl.ds(rows, PCHUNK)], sw[b])
            return carry

        lax.fori_loop(0, per_tile // 2, step, 0)
        for b in range(2):
            pltpu.make_async_copy(
                pbuf[b], out_hbm.at[pl.ds(chunk_rows(per_tile - 2 + b),
                                          PCHUNK)], sw[b]).wait()

    return pack_k


def _make_gather(n_rows):
    SUPER = 2 * CHUNK
    per_w = n_rows // NW
    n_chunks = per_w // CHUNK
    n_super = per_w // SUPER
    assert n_super % 2 == 0
    mesh = plsc.VectorSubcoreMesh(core_axis_name="c", subcore_axis_name="s")

    @functools.partial(
        pl.kernel,
        mesh=mesh,
        out_type=jax.ShapeDtypeStruct((n_rows // SUPER, SUPER, D),
                                      jnp.float32),
        compiler_params=_SC_PARAMS,
        scratch_types=[
            pltpu.VMEM((n_chunks, CHUNK), jnp.int32),
            pltpu.VMEM((SUPER, W), jnp.int32),
            pltpu.VMEM((SUPER, W), jnp.int32),
            pltpu.VMEM((SUPER, D), jnp.float32),
            pltpu.VMEM((SUPER, D), jnp.float32),
            pltpu.SemaphoreType.DMA,
            pltpu.SemaphoreType.DMA,
            pltpu.SemaphoreType.DMA,
            pltpu.SemaphoreType.DMA,
        ],
    )
    def gather_k(ids_hbm, table_hbm, out_hbm, idx_v, w0, w1, o0, o1,
                 sg0, sg1, sw0, sw1):
        wbuf, obuf = (w0, w1), (o0, o1)
        sg, sw = (sg0, sg1), (sw0, sw1)
        wid = lax.axis_index("s") * NC + lax.axis_index("c")
        base = wid * n_super
        pltpu.sync_copy(ids_hbm.at[pl.ds(wid * n_chunks, n_chunks)], idx_v)

        def start_gathers(j, b):
            pltpu.async_copy(table_hbm.at[idx_v.at[2 * j]],
                             wbuf[b].at[pl.ds(0, CHUNK)], sg[b])
            pltpu.async_copy(table_hbm.at[idx_v.at[2 * j + 1]],
                             wbuf[b].at[pl.ds(CHUNK, CHUNK)], sg[b])

        def wait_gathers(j, b):
            pltpu.make_async_copy(table_hbm.at[idx_v.at[2 * j]],
                                  wbuf[b].at[pl.ds(0, CHUNK)], sg[b]).wait()
            pltpu.make_async_copy(table_hbm.at[idx_v.at[2 * j + 1]],
                                  wbuf[b].at[pl.ds(CHUNK, CHUNK)],
                                  sg[b]).wait()

        start_gathers(0, 0)
        start_gathers(1, 1)

        def decode(src, dst):
            def row4(g, carry):
                r4 = pl.multiple_of(4 * g, 4)
                for k in range(4):
                    for v in range(2):
                        words = src[r4 + k, pl.ds(16 * v, L)]
                        for b in range(4):
                            vals = ((words >> (8 * b)) & 1).astype(
                                jnp.float32)
                            dst[r4 + k, pl.ds(64 * v + 16 * b, L)] = vals
                return carry
            lax.fori_loop(0, SUPER // 4, row4, 0)

        def group(g, carry):
            for b in range(2):
                j = 2 * g + b
                wait_gathers(j, b)
                @pl.when(g >= 1)
                def _():
                    pltpu.make_async_copy(
                        obuf[b], out_hbm.at[base + j], sw[b]).wait()
                decode(wbuf[b], obuf[b])
                @pl.when(g < n_super // 2 - 1)
                def _():
                    start_gathers(j + 2, b)
                pltpu.async_copy(obuf[b], out_hbm.at[base + j], sw[b])
            return carry

        lax.fori_loop(0, n_super // 2, group, 0)

        for b in range(2):
            pltpu.make_async_copy(
                obuf[b], out_hbm.at[base + n_super - 2 + b], sw[b]).wait()

    return gather_k


def kernel(input_ids, table):
    B, H = input_ids.shape
    V = table.shape[0]
    n_rows = B * H
    packed = _make_pack(V)(table)
    ids = input_ids.reshape(n_rows // CHUNK, CHUNK).astype(jnp.int32)
    out = _make_gather(n_rows)(ids, packed)
    return out.reshape(B, H, D)


# speedup vs baseline: 1.6166x; 1.6166x over previous
import functools

import jax
import jax.numpy as jnp
from jax import lax
from jax.experimental import pallas as pl
from jax.experimental.pallas import tpu as pltpu
from jax.experimental.pallas import tpu_sc as plsc

D = 128
W = D // 4
L = 16

_info = plsc.get_sparse_core_info()
NC, NS = _info.num_cores, _info.num_subcores
NW = NC * NS

CHUNK = 128
PCHUNK = 160

_SC_PARAMS = pltpu.CompilerParams(use_tc_tiling_on_sc=False)


def _make_pack(V):
    n_chunks = -(-V // PCHUNK)
    assert V % PCHUNK == 0
    mesh = plsc.VectorSubcoreMesh(core_axis_name="c", subcore_axis_name="s")

    @functools.partial(
        pl.kernel,
        mesh=mesh,
        out_type=jax.ShapeDtypeStruct((V, W), jnp.int32),
        compiler_params=_SC_PARAMS,
        scratch_types=[
            pltpu.VMEM((PCHUNK, D), jnp.float32),
            pltpu.VMEM((PCHUNK, D), jnp.float32),
            pltpu.VMEM((PCHUNK, W), jnp.int32),
            pltpu.VMEM((PCHUNK, W), jnp.int32),
            pltpu.SemaphoreType.DMA,
            pltpu.SemaphoreType.DMA,
            pltpu.SemaphoreType.DMA,
            pltpu.SemaphoreType.DMA,
        ],
    )
    def pack_k(tab_hbm, out_hbm, t0, t1, p0, p1, sg0, sg1, sw0, sw1):
        tbuf, pbuf = (t0, t1), (p0, p1)
        sg, sw = (sg0, sg1), (sw0, sw1)
        wid = lax.axis_index("s") * NC + lax.axis_index("c")
        per_tile = -(-n_chunks // NW)

        def chunk_rows(i):
            c = jnp.minimum(wid + i * NW, n_chunks - 1)
            return c * PCHUNK

        pltpu.async_copy(tab_hbm.at[pl.ds(chunk_rows(0), PCHUNK)], tbuf[0],
                         sg[0])
        pltpu.async_copy(tab_hbm.at[pl.ds(chunk_rows(1), PCHUNK)], tbuf[1],
                         sg[1])

        def encode(src, dst):
            def row(r, carry):
                for v in range(2):
                    word = None
                    for b in range(4):
                        x = src[r, pl.ds(64 * v + 16 * b, L)]
                        s = jnp.where(x >= 0, jnp.int32(1 << (8 * b)),
                                      jnp.int32(0))
                        word = s if word is None else word | s
                    dst[r, pl.ds(16 * v, L)] = word
                return carry
            lax.fori_loop(0, PCHUNK, row, 0)

        def step(i, carry):
            for b in range(2):
                g = 2 * i + b
                rows = chunk_rows(g)
                pltpu.make_async_copy(
                    tab_hbm.at[pl.ds(rows, PCHUNK)], tbuf[b], sg[b]).wait()
                @pl.when(g >= 2)
                def _():
                    pltpu.make_async_copy(
                        pbuf[b], out_hbm.at[pl.ds(rows, PCHUNK)], sw[b]).wait()
                encode(tbuf[b], pbuf[b])
                @pl.when(g + 2 < per_tile)
                def _():
                    pltpu.async_copy(
                        tab_hbm.at[pl.ds(chunk_rows(g + 2), PCHUNK)],
                        tbuf[b], sg[b])
                pltpu.async_copy(
                    pbuf[b], out_hbm.at[pl.ds(rows, PCHUNK)], sw[b])
            return carry

        lax.fori_loop(0, per_tile // 2, step, 0)
        for b in range(2):
            pltpu.make_async_copy(
                pbuf[b], out_hbm.at[pl.ds(chunk_rows(per_tile - 2 + b),
                                          PCHUNK)], sw[b]).wait()

    return pack_k


def _make_gather(n_rows):
    SUPER = 2 * CHUNK
    per_w = n_rows // NW
    n_chunks = per_w // CHUNK
    n_super = per_w // SUPER
    assert n_super % 2 == 0
    mesh = plsc.VectorSubcoreMesh(core_axis_name="c", subcore_axis_name="s")

    @functools.partial(
        pl.kernel,
        mesh=mesh,
        out_type=jax.ShapeDtypeStruct((n_rows // SUPER, SUPER, D),
                                      jnp.float32),
        compiler_params=_SC_PARAMS,
        scratch_types=[
            pltpu.VMEM((n_chunks, CHUNK), jnp.int32),
            pltpu.VMEM((SUPER, W), jnp.int32),
            pltpu.VMEM((SUPER, W), jnp.int32),
            pltpu.VMEM((SUPER, D), jnp.float32),
            pltpu.VMEM((SUPER, D), jnp.float32),
            pltpu.SemaphoreType.DMA,
            pltpu.SemaphoreType.DMA,
            pltpu.SemaphoreType.DMA,
            pltpu.SemaphoreType.DMA,
        ],
    )
    def gather_k(ids_hbm, table_hbm, out_hbm, idx_v, w0, w1, o0, o1,
                 sg0, sg1, sw0, sw1):
        wbuf, obuf = (w0, w1), (o0, o1)
        sg, sw = (sg0, sg1), (sw0, sw1)
        wid = lax.axis_index("s") * NC + lax.axis_index("c")
        base = wid * n_super
        pltpu.sync_copy(ids_hbm.at[pl.ds(wid * n_chunks, n_chunks)], idx_v)

        def start_gathers(j, b):
            pltpu.async_copy(table_hbm.at[idx_v.at[2 * j]],
                             wbuf[b].at[pl.ds(0, CHUNK)], sg[b])
            pltpu.async_copy(table_hbm.at[idx_v.at[2 * j + 1]],
                             wbuf[b].at[pl.ds(CHUNK, CHUNK)], sg[b])

        def wait_gathers(j, b):
            pltpu.make_async_copy(table_hbm.at[idx_v.at[2 * j]],
                                  wbuf[b].at[pl.ds(0, CHUNK)], sg[b]).wait()
            pltpu.make_async_copy(table_hbm.at[idx_v.at[2 * j + 1]],
                                  wbuf[b].at[pl.ds(CHUNK, CHUNK)],
                                  sg[b]).wait()

        start_gathers(0, 0)
        start_gathers(1, 1)

        def decode(src, dst):
            @functools.partial(plsc.parallel_loop, 0, SUPER, unroll=2)
            def row(r):
                for v in range(2):
                    words = src[r, pl.ds(16 * v, L)]
                    for b in range(4):
                        vals = ((words >> (8 * b)) & 1).astype(jnp.float32)
                        dst[r, pl.ds(64 * v + 16 * b, L)] = vals

        def group(g, carry):
            for b in range(2):
                j = 2 * g + b
                wait_gathers(j, b)
                @pl.when(g >= 1)
                def _():
                    pltpu.make_async_copy(
                        obuf[b], out_hbm.at[base + j], sw[b]).wait()
                decode(wbuf[b], obuf[b])
                @pl.when(g < n_super // 2 - 1)
                def _():
                    start_gathers(j + 2, b)
                pltpu.async_copy(obuf[b], out_hbm.at[base + j], sw[b])
            return carry

        lax.fori_loop(0, n_super // 2, group, 0)

        for b in range(2):
            pltpu.make_async_copy(
                obuf[b], out_hbm.at[base + n_super - 2 + b], sw[b]).wait()

    return gather_k


def kernel(input_ids, table):
    B, H = input_ids.shape
    V = table.shape[0]
    n_rows = B * H
    packed = _make_pack(V)(table)
    ids = input_ids.reshape(n_rows // CHUNK, CHUNK).astype(jnp.int32)
    out = _make_gather(n_rows)(ids, packed)
    return out.reshape(B, H, D)
